# Initial kernel scaffold; baseline (speedup 1.0000x reference)
import jax
import jax.numpy as jnp
from jax.experimental import pallas as pl


def kernel(proposal_boxes, gt_boxes, gt_classes):
    raise NotImplementedError("write your pallas kernel here")


# speedup vs baseline: 2.6135x; 2.6135x over previous
import jax
import jax.numpy as jnp
from jax import lax
from jax.experimental import pallas as pl
from jax.experimental.pallas import tpu as pltpu
from jax.experimental.pallas import tpu_sc as plsc

NUM_CLASSES = 80
IOU_THRESHOLD = 0.5
M = 100
N = 20000
LANES = 16
NW = 32
PPW = 640
NPAD = NW * PPW
VPW = PPW // LANES


def _roi_body(prop_hbm, gt_hbm, combo_hbm,
              vals_out, idxs_out, labs_out, cls_out,
              prop_v, gt_v, ga_v, combo_v,
              vals_v, idxs_v, labs_v, clso_v):
    c = lax.axis_index("c")
    s = lax.axis_index("s")
    wid = s * 2 + c
    base = wid * PPW

    pltpu.sync_copy(prop_hbm.at[:, pl.ds(base, PPW)], prop_v)
    pltpu.sync_copy(gt_hbm, gt_v)
    pltpu.sync_copy(combo_hbm, combo_v)

    def ga_body(g, carry):
        gx1 = gt_v[0, g, :]
        gy1 = gt_v[1, g, :]
        gx2 = gt_v[2, g, :]
        gy2 = gt_v[3, g, :]
        ga_v[g, :] = (gx2 - gx1) * (gy2 - gy1)
        return carry

    lax.fori_loop(0, M, ga_body, 0)

    def j_body(j, carry):
        o = j * LANES
        px1 = prop_v[0, pl.ds(o, LANES)]
        py1 = prop_v[1, pl.ds(o, LANES)]
        px2 = prop_v[2, pl.ds(o, LANES)]
        py2 = prop_v[3, pl.ds(o, LANES)]
        parea = (px2 - px1) * (py2 - py1)

        def g_body(g, st):
            binter, bd, bcombo = st
            gx1 = gt_v[0, g, :]
            gy1 = gt_v[1, g, :]
            gx2 = gt_v[2, g, :]
            gy2 = gt_v[3, g, :]
            ga = ga_v[g, :]
            combo = combo_v[g, :]
            w = jnp.maximum(jnp.minimum(px2, gx2) - jnp.maximum(px1, gx1), 0.0)
            h = jnp.maximum(jnp.minimum(py2, gy2) - jnp.maximum(py1, gy1), 0.0)
            inter = w * h
            d = jnp.maximum(parea + ga - inter, 1e-6)
            upd = inter * bd > binter * d
            binter = jnp.where(upd, inter, binter)
            bd = jnp.where(upd, d, bd)
            bcombo = jnp.where(upd, combo, bcombo)
            return binter, bd, bcombo

        init = (jnp.zeros((LANES,), jnp.float32),
                jnp.ones((LANES,), jnp.float32),
                combo_v[0, :])
        binter, bd, bcombo = lax.fori_loop(0, M, g_body, init)

        val = binter / bd
        fg = val >= IOU_THRESHOLD
        zero_i = jnp.zeros((LANES,), jnp.int32)
        bidx = lax.shift_right_logical(bcombo, 7)
        cls = lax.bitwise_and(bcombo, zero_i + 127)
        vals_v[pl.ds(o, LANES)] = val
        idxs_v[pl.ds(o, LANES)] = bidx
        labs_v[pl.ds(o, LANES)] = jnp.where(fg, zero_i + 1, zero_i)
        clso_v[pl.ds(o, LANES)] = jnp.where(fg, cls, zero_i + NUM_CLASSES)
        return carry

    lax.fori_loop(0, VPW, j_body, 0)

    pltpu.sync_copy(vals_v, vals_out.at[pl.ds(base, PPW)])
    pltpu.sync_copy(idxs_v, idxs_out.at[pl.ds(base, PPW)])
    pltpu.sync_copy(labs_v, labs_out.at[pl.ds(base, PPW)])
    pltpu.sync_copy(clso_v, cls_out.at[pl.ds(base, PPW)])


_roi = pl.kernel(
    _roi_body,
    out_type=(jax.ShapeDtypeStruct((NPAD,), jnp.float32),
              jax.ShapeDtypeStruct((NPAD,), jnp.int32),
              jax.ShapeDtypeStruct((NPAD,), jnp.int32),
              jax.ShapeDtypeStruct((NPAD,), jnp.int32)),
    mesh=plsc.VectorSubcoreMesh(core_axis_name="c", subcore_axis_name="s"),
    scratch_types=[
        pltpu.VMEM((4, PPW), jnp.float32),
        pltpu.VMEM((4, M, LANES), jnp.float32),
        pltpu.VMEM((M, LANES), jnp.float32),
        pltpu.VMEM((M, LANES), jnp.int32),
        pltpu.VMEM((PPW,), jnp.float32),
        pltpu.VMEM((PPW,), jnp.int32),
        pltpu.VMEM((PPW,), jnp.int32),
        pltpu.VMEM((PPW,), jnp.int32),
    ],
)


def kernel(proposal_boxes, gt_boxes, gt_classes):
    prop_t = jnp.transpose(proposal_boxes.astype(jnp.float32))
    prop_t = jnp.pad(prop_t, ((0, 0), (0, NPAD - N)))
    gt_rep = jnp.broadcast_to(
        jnp.transpose(gt_boxes.astype(jnp.float32))[:, :, None], (4, M, LANES))
    combo = (jnp.arange(M, dtype=jnp.int32) * 128
             + gt_classes.astype(jnp.int32))
    combo_rep = jnp.broadcast_to(combo[:, None], (M, LANES))
    vals, idxs, labs, cls = _roi(prop_t, gt_rep, combo_rep)
    return vals[:N], idxs[:N], labs[:N], cls[:N]
